# Initial kernel scaffold; baseline (speedup 1.0000x reference)
import jax
import jax.numpy as jnp
from jax.experimental import pallas as pl


def kernel(input_ids, attention_mask, wte, wpe):
    raise NotImplementedError("write your pallas kernel here")


# speedup vs baseline: 1.0314x; 1.0314x over previous
import functools

import jax
import jax.numpy as jnp
from jax import lax
from jax.experimental import pallas as pl
from jax.experimental.pallas import tpu as pltpu
from jax.experimental.pallas import tpu_sc as plsc

D = 768
LANES = 16
ROW_V = D // LANES

NC = 2
NS = 16
NW = NC * NS


def _make_emb_kernel(BS: int, S: int):
    per_w = BS // NW
    nchunk = 4
    chunk = per_w // nchunk

    mesh = plsc.VectorSubcoreMesh(core_axis_name="c", subcore_axis_name="s")

    @functools.partial(
        pl.kernel,
        mesh=mesh,
        out_type=[
            jax.ShapeDtypeStruct((BS, D), jnp.float32),
            jax.ShapeDtypeStruct((BS,), jnp.float32),
        ],
        scratch_types=[
            pltpu.VMEM((nchunk, chunk), jnp.int32),
            pltpu.VMEM((chunk, D), jnp.float32),
            pltpu.VMEM((chunk, D), jnp.float32),
            pltpu.VMEM((per_w,), jnp.float32),
            pltpu.VMEM((per_w,), jnp.float32),
            pltpu.SemaphoreType.DMA,
            pltpu.SemaphoreType.DMA,
        ],
    )
    def emb_kernel(ids_hbm, mask_hbm, wte_hbm, wpe_hbm,
                   out_hbm, am_hbm,
                   idx_v, tok_buf, pos_buf, mask_v, am_v, sem_t, sem_p):
        wid = lax.axis_index("s") * NC + lax.axis_index("c")
        base = wid * per_w

        pltpu.sync_copy(ids_hbm.at[wid], idx_v)
        pltpu.sync_copy(mask_hbm.at[pl.ds(base, per_w)], mask_v)

        neg_inf = jnp.float32(jnp.finfo(jnp.float32).min)
        for i in range(per_w // LANES):
            m = mask_v[pl.ds(i * LANES, LANES)]
            am_v[pl.ds(i * LANES, LANES)] = (1.0 - m) * neg_inf
        pltpu.sync_copy(am_v, am_hbm.at[pl.ds(base, per_w)])

        pos_base = lax.rem(base, S)

        for c in range(nchunk):
            row0 = base + c * chunk
            p0 = pos_base + c * chunk
            cp_t = pltpu.async_copy(wte_hbm.at[idx_v.at[c]], tok_buf, sem_t)
            cp_p = pltpu.async_copy(wpe_hbm.at[pl.ds(p0, chunk)], pos_buf, sem_p)
            cp_t.wait()
            cp_p.wait()

            def add_row(r, carry):
                for j in range(ROW_V):
                    sl = pl.ds(j * LANES, LANES)
                    tok_buf[r, sl] += pos_buf[r, sl]
                return carry

            lax.fori_loop(0, chunk, add_row, 0)
            pltpu.sync_copy(tok_buf, out_hbm.at[pl.ds(row0, chunk)])

    return emb_kernel


def kernel(input_ids, attention_mask, wte, wpe):
    input_shape = input_ids.shape
    S = input_shape[-1]
    ids2 = input_ids.reshape(-1, S)
    batch = ids2.shape[0]
    BS = batch * S

    per_w = BS // NW
    nchunk = 4
    chunk = per_w // nchunk

    ids3 = ids2.reshape(NW, nchunk, chunk).astype(jnp.int32)
    mask_flat = attention_mask.reshape(BS).astype(jnp.float32)

    hidden, am = _make_emb_kernel(BS, S)(ids3, mask_flat, wte, wpe)
    hidden = hidden.reshape(batch, S, D)
    am = am.reshape(batch, 1, 1, S)
    return (hidden, am)
